# baseline (speedup 1.0000x reference)
import jax
import jax.numpy as jnp
from jax import lax
from jax.experimental import pallas as pl
from jax.experimental.pallas import tpu as pltpu
from jax.experimental.pallas import tpu_sc as plsc

N = 10000
D = 128
E = 320000
V = 3
LAMBD = 0.01

NC = 2
NS = 16
NW = NC * NS
L = 16

NPAD = 10240
C = 128
NCHUNK = 80
HC = NCHUNK // 2
NP = HC // 2
EPAD = NW * NCHUNK * C
RPT = NPAD // NS


def _encoder(features, Ws, bs):
    B = 1280
    G = NPAD // B

    def body(f_ref, w0, b0, w1, b1, w2, b2, o0, o1, o2):
        x = f_ref[...]
        for w, b, o in ((w0, b0, o0), (w1, b1, o1), (w2, b2, o2)):
            o[...] = lax.dot_general(
                x, w[...], (((1,), (0,)), ((), ())),
                preferred_element_type=jnp.float32) + b[...]

    mat = pl.BlockSpec((D, D), lambda k: (0, 0))
    vec = pl.BlockSpec((1, D), lambda k: (0, 0))
    blk = pl.BlockSpec((B, D), lambda k: (k, 0))
    return pl.pallas_call(
        body,
        grid=(G,),
        in_specs=[blk, mat, vec, mat, vec, mat, vec],
        out_specs=[blk] * V,
        out_shape=[jax.ShapeDtypeStruct((NPAD, D), jnp.float32)] * V,
    )(features, Ws[0], bs[0], Ws[1], bs[1], Ws[2], bs[2])


def _sc_body(ha0, ha1, ha2, dst0, src0, ew0, dst1, src1, ew1, dst2, src2,
             ew2, p0, p1, p2, dst_v, src_v, ew_v, buf0, buf1, accum,
             sg0, sg1, ss0, ss1):
    cid = lax.axis_index("c")
    sid = lax.axis_index("s")
    wid = sid * NC + cid
    base_row = sid * RPT

    zvec = jnp.zeros((L,), jnp.float32)
    bufs = (buf0, buf1)
    gsems = (sg0, sg1)
    ssems = (ss0, ss1)

    views = ((ha0, dst0, src0, ew0, p0), (ha1, dst1, src1, ew1, p1),
             (ha2, dst2, src2, ew2, p2))
    for ha, dst_h, src_h, ew_h, out in views:
        def zrow(r, carry):
            for k in range(D // L):
                buf0[r, pl.ds(k * L, L)] = zvec
            return carry

        lax.fori_loop(0, C, zrow, 0)
        for q in range(RPT // C):
            pltpu.sync_copy(buf0, accum.at[pl.ds(base_row + q * C, C)])
        plsc.subcore_barrier()

        def gather_start(j, b):
            pltpu.async_copy(ha.at[src_v.at[j]], bufs[b], gsems[b])

        def gather_wait(j, b):
            pltpu.make_async_copy(ha.at[src_v.at[j]], bufs[b],
                                  gsems[b]).wait()

        def scatter_start(j, b):
            pltpu.async_copy(bufs[b], accum.at[dst_v.at[j]], ssems[b],
                             add=True)

        def scatter_wait(j, b):
            pltpu.make_async_copy(bufs[b], accum.at[dst_v.at[j]],
                                  ssems[b]).wait()

        def scale(j, b):
            buf = bufs[b]

            def srow(r16, carry):
                w_all = ew_v[j, pl.ds(r16 * L, L)]
                for q0 in range(0, L, 4):
                    ws = [jnp.full((L,), w_all[q0 + t], jnp.float32)
                          for t in range(4)]
                    rows = [r16 * L + q0 + t for t in range(4)]
                    vals = [[buf[r, pl.ds(k * L, L)]
                             for k in range(D // L)] for r in rows]
                    for t, r in enumerate(rows):
                        for k in range(D // L):
                            buf[r, pl.ds(k * L, L)] = vals[t][k] * ws[t]
                return carry

            lax.fori_loop(0, C // L, srow, 0)

        def run_half(hh, carry):
            off = pl.multiple_of(hh * HC, 8)
            pltpu.sync_copy(dst_h.at[wid, pl.ds(off, HC)], dst_v)
            pltpu.sync_copy(src_h.at[wid, pl.ds(off, HC)], src_v)
            pltpu.sync_copy(ew_h.at[wid, pl.ds(off, HC)], ew_v)
            gather_start(0, 0)
            gather_start(1, 1)

            def pair(p, pcarry):
                j0 = p * 2
                j1 = j0 + 1
                gather_wait(j0, 0)
                scale(j0, 0)
                scatter_start(j0, 0)
                gather_wait(j1, 1)
                scale(j1, 1)
                scatter_start(j1, 1)

                @pl.when(p < NP - 1)
                def _():
                    scatter_wait(j0, 0)
                    gather_start(j0 + 2, 0)
                    scatter_wait(j1, 1)
                    gather_start(j1 + 2, 1)

                @pl.when(p == NP - 1)
                def _():
                    scatter_wait(j0, 0)
                    scatter_wait(j1, 1)

                return pcarry

            lax.fori_loop(0, NP, pair, 0)
            return carry

        lax.fori_loop(0, 2, run_half, 0)
        plsc.subcore_barrier()
        pltpu.sync_copy(accum.at[pl.ds(base_row, RPT)],
                        out.at[cid, pl.ds(base_row, RPT)])


def _sc_spmm(ha, edges):
    mesh = plsc.VectorSubcoreMesh(core_axis_name="c", subcore_axis_name="s",
                                  num_cores=NC, num_subcores=NS)
    run = pl.kernel(
        _sc_body,
        out_type=[jax.ShapeDtypeStruct((NC, NPAD, D), jnp.float32)] * V,
        mesh=mesh,
        compiler_params=pltpu.CompilerParams(use_tc_tiling_on_sc=False),
        scratch_types=[
            pltpu.VMEM((HC, C), jnp.int32),
            pltpu.VMEM((HC, C), jnp.int32),
            pltpu.VMEM((HC, C), jnp.float32),
            pltpu.VMEM((C, D), jnp.float32),
            pltpu.VMEM((C, D), jnp.float32),
            pltpu.VMEM_SHARED((NPAD, D), jnp.float32),
            pltpu.SemaphoreType.DMA,
            pltpu.SemaphoreType.DMA,
            pltpu.SemaphoreType.DMA,
            pltpu.SemaphoreType.DMA,
        ],
    )
    args = list(ha)
    for d, s, w in edges:
        args.extend((d, s, w))
    return run(*args)


def _loss(ha, parts):
    B = 1280
    G = NPAD // B

    def body(a0, a1, a2, p00, p01, p10, p11, p20, p21, out,
             ci0, ci1, ci2, cx0, cx1, cx2):
        k = pl.program_id(0)
        has = (a0[...], a1[...], a2[...])
        hps = tuple(lo[0] + hi[0] for lo, hi in
                    ((p00, p01), (p10, p11), (p20, p21)))
        accs = (ci0, ci1, ci2, cx0, cx1, cx2)
        dn = (((0,), (0,)), ((), ()))
        terms = [lax.dot_general(hps[i], has[i], dn,
                                 preferred_element_type=jnp.float32)
                 for i in range(V)]
        terms += [lax.dot_general(hps[i], hps[(i + 1) % V], dn,
                                  preferred_element_type=jnp.float32)
                  for i in range(V)]

        @pl.when(k == 0)
        def _():
            for acc, t in zip(accs, terms):
                acc[...] = t

        @pl.when(k > 0)
        def _():
            for acc, t in zip(accs, terms):
                acc[...] = acc[...] + t

        @pl.when(k == G - 1)
        def _():
            ii = lax.broadcasted_iota(jnp.int32, (D, D), 0)
            jj = lax.broadcasted_iota(jnp.int32, (D, D), 1)
            eye = ii == jj
            loss = jnp.float32(0.0)
            for acc in accs:
                c = acc[...]
                on = jnp.sum(jnp.where(eye, (c - 1.0) ** 2, 0.0))
                off = jnp.sum(jnp.where(eye, 0.0, c * c))
                loss = loss + on + LAMBD * off
            out[...] = jnp.reshape(loss, (1, 1))

    blk = pl.BlockSpec((B, D), lambda k: (k, 0))
    lo = pl.BlockSpec((1, B, D), lambda k: (0, k, 0))
    hi = pl.BlockSpec((1, B, D), lambda k: (1, k, 0))
    return pl.pallas_call(
        body,
        grid=(G,),
        in_specs=[blk, blk, blk, lo, hi, lo, hi, lo, hi],
        out_specs=pl.BlockSpec((1, 1), lambda k: (0, 0)),
        out_shape=jax.ShapeDtypeStruct((1, 1), jnp.float32),
        scratch_shapes=[pltpu.VMEM((D, D), jnp.float32)] * 6,
    )(ha[0], ha[1], ha[2], parts[0], parts[0], parts[1], parts[1],
      parts[2], parts[2])


def _prep_edges(ei, ew):
    pad = EPAD - E
    dst = jnp.concatenate([ei[0], jnp.zeros((pad,), jnp.int32)])
    src = jnp.concatenate([ei[1], jnp.zeros((pad,), jnp.int32)])
    eww = jnp.concatenate([ew, jnp.zeros((pad,), jnp.float32)])
    return (dst.reshape(NW, NCHUNK, C), src.reshape(NW, NCHUNK, C),
            eww.reshape(NW, NCHUNK, C))


def kernel(features, W0, b0, W1, b1, W2, b2, edge_index_0, edge_weight_0,
           edge_index_1, edge_weight_1, edge_index_2, edge_weight_2):
    Ws = (W0, W1, W2)
    bs = (b0.reshape(1, D), b1.reshape(1, D), b2.reshape(1, D))
    fpad = jnp.concatenate(
        [features, jnp.zeros((NPAD - N, D), jnp.float32)])
    ha = _encoder(fpad, Ws, bs)
    edges = [_prep_edges(ei, ew) for ei, ew in
             ((edge_index_0, edge_weight_0), (edge_index_1, edge_weight_1),
              (edge_index_2, edge_weight_2))]
    parts = _sc_spmm(ha, edges)
    out = _loss(ha, parts)
    return jnp.reshape(out, ())


# speedup vs baseline: 1.6915x; 1.6915x over previous
import jax
import jax.numpy as jnp
from jax import lax
from jax.experimental import pallas as pl
from jax.experimental.pallas import tpu as pltpu
from jax.experimental.pallas import tpu_sc as plsc

N = 10000
D = 128
E = 320000
V = 3
LAMBD = 0.01

NC = 2
NS = 16
L = 16
D2 = D // NC

NPAD = 10240
C = 128
EPT = -(-E // NS)
NCHUNK = -(-EPT // C)
if NCHUNK % 2:
    NCHUNK += 1
NP = NCHUNK // 2
EPAD = NS * NCHUNK * C
ZR = 128
RPT = NPAD // NS


def _encoder(features, Ws, bs):
    B = 1280
    G = NPAD // B

    def body(f_ref, w0, b0, w1, b1, w2, b2, *outs):
        x = f_ref[...]
        for i, (w, b) in enumerate(((w0, b0), (w1, b1), (w2, b2))):
            r = lax.dot_general(
                x, w[...], (((1,), (0,)), ((), ())),
                preferred_element_type=jnp.float32) + b[...]
            outs[2 * i][...] = lax.slice(r, (0, 0), (B, D2))
            outs[2 * i + 1][...] = lax.slice(r, (0, D2), (B, D))

    mat = pl.BlockSpec((D, D), lambda k: (0, 0))
    vec = pl.BlockSpec((1, D), lambda k: (0, 0))
    blk = pl.BlockSpec((B, D), lambda k: (k, 0))
    half = pl.BlockSpec((B, D2), lambda k: (k, 0))
    return pl.pallas_call(
        body,
        grid=(G,),
        in_specs=[blk, mat, vec, mat, vec, mat, vec],
        out_specs=[half] * (V * NC),
        out_shape=[jax.ShapeDtypeStruct((NPAD, D2), jnp.float32)] * (V * NC),
    )(features, Ws[0], bs[0], Ws[1], bs[1], Ws[2], bs[2])


def _sc_body(a0l, a0h, a1l, a1h, a2l, a2h, dst0, src0, ew0, dst1, src1, ew1,
             dst2, src2, ew2, p0l, p0h, p1l, p1h, p2l, p2h,
             dst_v, src_v, ew_v, buf0, buf1, zb, accum, sg0, sg1, ss0, ss1):
    cid = lax.axis_index("c")
    sid = lax.axis_index("s")
    base_row = sid * RPT

    zvec = jnp.zeros((L,), jnp.float32)

    def zrow(r, carry):
        for k in range(D2 // L):
            zb[r, pl.ds(k * L, L)] = zvec
        return carry

    lax.fori_loop(0, ZR, zrow, 0)

    bufs = (buf0, buf1)
    gsems = (sg0, sg1)
    ssems = (ss0, ss1)

    views = (((a0l, a0h), dst0, src0, ew0, (p0l, p0h)),
             ((a1l, a1h), dst1, src1, ew1, (p1l, p1h)),
             ((a2l, a2h), dst2, src2, ew2, (p2l, p2h)))
    for vi, (ha_halves, dst_h, src_h, ew_h, out_halves) in enumerate(views):
        with jax.named_scope(f"zero{vi}"):
            for q in range(RPT // ZR):
                pltpu.sync_copy(zb, accum.at[pl.ds(base_row + q * ZR, ZR)])
        with jax.named_scope(f"slab{vi}"):
            pltpu.sync_copy(dst_h.at[sid], dst_v)
            pltpu.sync_copy(src_h.at[sid], src_v)
            pltpu.sync_copy(ew_h.at[sid], ew_v)
            plsc.subcore_barrier()

        for half in range(NC):
            ha = ha_halves[half]

            def gather_start(j, b):
                pltpu.async_copy(ha.at[src_v.at[j]], bufs[b], gsems[b])

            def gather_wait(j, b):
                pltpu.make_async_copy(ha.at[src_v.at[j]], bufs[b],
                                      gsems[b]).wait()

            def scatter_start(j, b):
                pltpu.async_copy(bufs[b], accum.at[dst_v.at[j]], ssems[b],
                                 add=True)

            def scatter_wait(j, b):
                pltpu.make_async_copy(bufs[b], accum.at[dst_v.at[j]],
                                      ssems[b]).wait()

            def scale(j, b):
                buf = bufs[b]

                def srow(r16, carry):
                    w_all = ew_v[j, pl.ds(r16 * L, L)]
                    for q0 in range(0, L, 4):
                        ws = [jnp.full((L,), w_all[q0 + t], jnp.float32)
                              for t in range(4)]
                        rows = [r16 * L + q0 + t for t in range(4)]
                        vals = [[buf[r, pl.ds(k * L, L)]
                                 for k in range(D2 // L)] for r in rows]
                        for t, r in enumerate(rows):
                            for k in range(D2 // L):
                                buf[r, pl.ds(k * L, L)] = vals[t][k] * ws[t]
                    return carry

                lax.fori_loop(0, C // L, srow, 0)

            def pair(p, carry):
                j0 = p * 2
                j1 = j0 + 1
                gather_wait(j0, 0)
                scale(j0, 0)
                scatter_start(j0, 0)
                gather_wait(j1, 1)
                scale(j1, 1)
                scatter_start(j1, 1)

                @pl.when(p < NP - 1)
                def _():
                    scatter_wait(j0, 0)
                    gather_start(j0 + 2, 0)
                    scatter_wait(j1, 1)
                    gather_start(j1 + 2, 1)

                @pl.when(p == NP - 1)
                def _():
                    scatter_wait(j0, 0)
                    scatter_wait(j1, 1)

                return carry

            @pl.when(cid == half)
            def _():
                with jax.named_scope(f"pipe{vi}"):
                    gather_start(0, 0)
                    gather_start(1, 1)
                    lax.fori_loop(0, NP, pair, 0)

        plsc.subcore_barrier()
        with jax.named_scope(f"copyout{vi}"):
            for half in range(NC):
                @pl.when(cid == half)
                def _(out=out_halves[half]):
                    pltpu.sync_copy(accum.at[pl.ds(base_row, RPT)],
                                    out.at[pl.ds(base_row, RPT)])


def _sc_spmm(ha, edges):
    mesh = plsc.VectorSubcoreMesh(core_axis_name="c", subcore_axis_name="s",
                                  num_cores=NC, num_subcores=NS)
    run = pl.kernel(
        _sc_body,
        out_type=[jax.ShapeDtypeStruct((NPAD, D2), jnp.float32)] * (V * NC),
        mesh=mesh,
        compiler_params=pltpu.CompilerParams(use_tc_tiling_on_sc=False),
        scratch_types=[
            pltpu.VMEM((NCHUNK, C), jnp.int32),
            pltpu.VMEM((NCHUNK, C), jnp.int32),
            pltpu.VMEM((NCHUNK, C), jnp.float32),
            pltpu.VMEM((C, D2), jnp.float32),
            pltpu.VMEM((C, D2), jnp.float32),
            pltpu.VMEM((ZR, D2), jnp.float32),
            pltpu.VMEM_SHARED((NPAD, D2), jnp.float32),
            pltpu.SemaphoreType.DMA,
            pltpu.SemaphoreType.DMA,
            pltpu.SemaphoreType.DMA,
            pltpu.SemaphoreType.DMA,
        ],
    )
    args = list(ha)
    for d, s, w in edges:
        args.extend((d, s, w))
    return run(*args)


def _loss(ha, parts):
    B = 1280
    G = NPAD // B

    def body(a0l, a0h, a1l, a1h, a2l, a2h, p0l, p0h, p1l, p1h, p2l, p2h,
             out, ci0, ci1, ci2, cx0, cx1, cx2):
        k = pl.program_id(0)
        has = tuple(
            jnp.concatenate((lo[...], hi[...]), axis=1)
            for lo, hi in ((a0l, a0h), (a1l, a1h), (a2l, a2h)))
        hps = tuple(
            jnp.concatenate((lo[...], hi[...]), axis=1)
            for lo, hi in ((p0l, p0h), (p1l, p1h), (p2l, p2h)))
        accs = (ci0, ci1, ci2, cx0, cx1, cx2)
        dn = (((0,), (0,)), ((), ()))
        terms = [lax.dot_general(hps[i], has[i], dn,
                                 preferred_element_type=jnp.float32)
                 for i in range(V)]
        terms += [lax.dot_general(hps[i], hps[(i + 1) % V], dn,
                                  preferred_element_type=jnp.float32)
                  for i in range(V)]

        @pl.when(k == 0)
        def _():
            for acc, t in zip(accs, terms):
                acc[...] = t

        @pl.when(k > 0)
        def _():
            for acc, t in zip(accs, terms):
                acc[...] = acc[...] + t

        @pl.when(k == G - 1)
        def _():
            ii = lax.broadcasted_iota(jnp.int32, (D, D), 0)
            jj = lax.broadcasted_iota(jnp.int32, (D, D), 1)
            eye = ii == jj
            loss = jnp.float32(0.0)
            for acc in accs:
                c = acc[...]
                on = jnp.sum(jnp.where(eye, (c - 1.0) ** 2, 0.0))
                off = jnp.sum(jnp.where(eye, 0.0, c * c))
                loss = loss + on + LAMBD * off
            out[...] = jnp.reshape(loss, (1, 1))

    half = pl.BlockSpec((B, D2), lambda k: (k, 0))
    return pl.pallas_call(
        body,
        grid=(G,),
        in_specs=[half] * 12,
        out_specs=pl.BlockSpec((1, 1), lambda k: (0, 0)),
        out_shape=jax.ShapeDtypeStruct((1, 1), jnp.float32),
        scratch_shapes=[pltpu.VMEM((D, D), jnp.float32)] * 6,
    )(*ha, *parts)


def _prep_edges(ei, ew):
    pad = EPAD - E
    dst = jnp.concatenate([ei[0], jnp.zeros((pad,), jnp.int32)])
    src = jnp.concatenate([ei[1], jnp.zeros((pad,), jnp.int32)])
    eww = jnp.concatenate([ew, jnp.zeros((pad,), jnp.float32)])
    return (dst.reshape(NS, NCHUNK, C), src.reshape(NS, NCHUNK, C),
            eww.reshape(NS, NCHUNK, C))


def kernel(features, W0, b0, W1, b1, W2, b2, edge_index_0, edge_weight_0,
           edge_index_1, edge_weight_1, edge_index_2, edge_weight_2):
    Ws = (W0, W1, W2)
    bs = (b0.reshape(1, D), b1.reshape(1, D), b2.reshape(1, D))
    fpad = jnp.concatenate(
        [features, jnp.zeros((NPAD - N, D), jnp.float32)])
    ha = _encoder(fpad, Ws, bs)
    edges = [_prep_edges(ei, ew) for ei, ew in
             ((edge_index_0, edge_weight_0), (edge_index_1, edge_weight_1),
              (edge_index_2, edge_weight_2))]
    parts = _sc_spmm(ha, edges)
    out = _loss(ha, parts)
    return jnp.reshape(out, ())
